# baseline (speedup 1.0000x reference)
import functools

import jax
import jax.numpy as jnp
from jax import lax
from jax.experimental import pallas as pl
from jax.experimental.pallas import tpu as pltpu
from jax.experimental.pallas import tpu_sc as plsc

N = 10000
E = 320000
D = 128

NC = 2
NS = 16
NW = NC * NS
EPW = E // NW
C = 80
NCH = EPW // C
RPT = 624
ZR = 16

RB = 1000


NI = 8
NR = 4
NF = 2
UNR = 4
GDEP = 2
IDEP = 4


def _sc_spmv_body(sup_hbm, src_hbm, dst_hbm, val_hbm, out_hbm,
                  acc_sh, zbuf, *bufs):
    IS = list(bufs[0:NI])
    ID = list(bufs[NI:2 * NI])
    VV = list(bufs[2 * NI:3 * NI])
    RW = list(bufs[3 * NI:3 * NI + NR])
    RF = list(bufs[3 * NI + NR:3 * NI + NR + NF])
    o = 3 * NI + NR + NF
    SI = list(bufs[o:o + NI])
    SG = list(bufs[o + NI:o + NI + NR])
    SS = list(bufs[o + NI + NR:o + NI + NR + NF])

    c = lax.axis_index("c")
    s = lax.axis_index("s")
    wid = c * NS + s

    zeros16 = jnp.zeros((16,), jnp.float32)
    for i in range(ZR):
        for j in range(D // 16):
            zbuf[i, pl.ds(j * 16, 16)] = zeros16

    def zero_body(k, carry):
        pltpu.sync_copy(zbuf, acc_sh.at[pl.ds(s * RPT + k * ZR, ZR)])
        return carry

    nz = RPT // ZR + jnp.where(s == NS - 1, (N - NS * RPT) // ZR, 0)
    lax.fori_loop(0, nz, zero_body, 0)
    plsc.subcore_barrier()

    base = wid * EPW

    def idx_start(t, b):
        off = base + t * C
        pltpu.async_copy(src_hbm.at[pl.ds(off, C)], IS[b], SI[b])
        pltpu.async_copy(dst_hbm.at[pl.ds(off, C)], ID[b], SI[b])
        pltpu.async_copy(val_hbm.at[pl.ds(off, C)], VV[b], SI[b])

    def idx_wait(b):
        pltpu.make_async_copy(src_hbm.at[pl.ds(0, C)], IS[b], SI[b]).wait()
        pltpu.make_async_copy(dst_hbm.at[pl.ds(0, C)], ID[b], SI[b]).wait()
        pltpu.make_async_copy(val_hbm.at[pl.ds(0, C)], VV[b], SI[b]).wait()

    def gather_start(bi, br):
        pltpu.async_copy(sup_hbm.at[IS[bi]], RW[br], SG[br])

    def gather_wait(bi, br):
        pltpu.make_async_copy(sup_hbm.at[IS[bi]], RW[br], SG[br]).wait()

    def scatter_start(bi, bf):
        pltpu.async_copy(RF[bf], acc_sh.at[ID[bi]], SS[bf], add=True)

    def scatter_wait(bi, bf):
        pltpu.make_async_copy(RF[bf], acc_sh.at[ID[bi]], SS[bf]).wait()

    HIMASK = jnp.full((16,), -65536, jnp.int32)

    def scale(bi, br, bf):
        def ebody(e, iv):
            for u in range(UNR):
                ee = e * UNR + u
                bc = plsc.load_gather(VV[bi], [iv + u])
                for j in range(D // 32):
                    v = RW[br][ee, pl.ds(j * 16, 16)]
                    ev = plsc.bitcast(lax.shift_left(v, 16), jnp.float32)
                    od = plsc.bitcast(jnp.bitwise_and(v, HIMASK),
                                      jnp.float32)
                    RF[bf][ee, pl.ds(j * 32, 16)] = ev * bc
                    RF[bf][ee, pl.ds(j * 32 + 16, 16)] = od * bc
            return iv + UNR

        lax.fori_loop(0, C // UNR, ebody, jnp.zeros((16,), jnp.int32))

    def body(t, ph, wait_scatter=True, do_idx=True, do_gather=True):
        bi, br, bf = ph % NI, ph % NR, ph % NF
        if wait_scatter:
            scatter_wait((ph - GDEP) % NI, (ph - GDEP) % NF)
        if do_idx:
            idx_start(t + IDEP, (ph + IDEP) % NI)
        if do_gather:
            idx_wait((ph + GDEP) % NI)
            gather_start((ph + GDEP) % NI, (ph + GDEP) % NR)
        gather_wait(bi, br)
        scale(bi, br, bf)
        scatter_start(bi, bf)

    for t in range(IDEP):
        idx_start(t, t)
    for t in range(GDEP):
        idx_wait(t)
        gather_start(t, t)
    for t in range(IDEP):
        body(t, t, wait_scatter=(t >= GDEP))

    NGRP = (NCH - IDEP - IDEP) // NI
    T0 = IDEP

    def group(p, carry):
        t0 = T0 + p * NI
        for u in range(NI):
            body(t0 + u, T0 + u)
        return carry

    lax.fori_loop(0, NGRP, group, 0)

    for t in range(T0 + NGRP * NI, NCH):
        body(t, t,
             do_idx=(t + IDEP <= NCH - 1),
             do_gather=(t + GDEP <= NCH - 1))
    for t in range(NCH - GDEP, NCH):
        scatter_wait(t % NI, t % NF)

    plsc.subcore_barrier()
    pltpu.sync_copy(acc_sh.at[pl.ds(s * RPT, RPT)],
                    out_hbm.at[c, pl.ds(s * RPT, RPT)])

    @pl.when(s == NS - 1)
    def _tail():
        pltpu.sync_copy(acc_sh.at[pl.ds(NS * RPT, N - NS * RPT)],
                        out_hbm.at[c, pl.ds(NS * RPT, N - NS * RPT)])


_sc_spmv = pl.kernel(
    _sc_spmv_body,
    out_type=jax.ShapeDtypeStruct((NC, N, D), jnp.float32),
    mesh=plsc.VectorSubcoreMesh(core_axis_name="c", subcore_axis_name="s",
                                num_cores=NC, num_subcores=NS),
    scratch_types=(
        [pltpu.MemorySpace.VMEM_SHARED((N, D), jnp.float32),
         pltpu.VMEM((ZR, D), jnp.float32)]
        + [pltpu.VMEM((C,), jnp.int32) for _ in range(2 * NI)]
        + [pltpu.VMEM((C,), jnp.float32) for _ in range(NI)]
        + [pltpu.VMEM((C, D // 2), jnp.int32) for _ in range(NR)]
        + [pltpu.VMEM((C, D), jnp.float32) for _ in range(NF)]
        + [pltpu.SemaphoreType.DMA for _ in range(NI + NR + NF)]
    ),
    compiler_params=pltpu.CompilerParams(needs_layout_passes=False,
                                         use_tc_tiling_on_sc=False),
)


def _mm_plain_body(x_ref, w_ref, o_ref):
    o_ref[...] = jnp.dot(x_ref[...], w_ref[...],
                         preferred_element_type=jnp.float32
                         ).astype(jnp.bfloat16)


def _mm_plain(x, W):
    return pl.pallas_call(
        _mm_plain_body,
        grid=(N // RB,),
        in_specs=[pl.BlockSpec((RB, D), lambda i: (i, 0)),
                  pl.BlockSpec((D, D), lambda i: (0, 0))],
        out_specs=pl.BlockSpec((RB, D), lambda i: (i, 0)),
        out_shape=jax.ShapeDtypeStruct((N, D), jnp.bfloat16),
    )(x, W)


def _mm_fused_body(p_ref, b_ref, w_ref, o_ref):
    h = jnp.maximum(p_ref[0] + p_ref[1] + b_ref[...], 0.0)
    o_ref[...] = jnp.dot(h, w_ref[...],
                         preferred_element_type=jnp.float32
                         ).astype(jnp.bfloat16)


def _mm_fused(p, b, W):
    return pl.pallas_call(
        _mm_fused_body,
        grid=(N // RB,),
        in_specs=[pl.BlockSpec((NC, RB, D), lambda i: (0, i, 0)),
                  pl.BlockSpec((1, D), lambda i: (0, 0)),
                  pl.BlockSpec((D, D), lambda i: (0, 0))],
        out_specs=pl.BlockSpec((RB, D), lambda i: (i, 0)),
        out_shape=jax.ShapeDtypeStruct((N, D), jnp.bfloat16),
    )(p, b.reshape(1, D), W)


def _final_body(p_ref, b_ref, m_ref, o_ref):
    h = p_ref[0] + p_ref[1]
    o_ref[...] = jnp.dot(h, m_ref[...],
                         preferred_element_type=jnp.float32) + b_ref[...]


def _final(p, b, M):
    return pl.pallas_call(
        _final_body,
        grid=(N // RB,),
        in_specs=[pl.BlockSpec((NC, RB, D), lambda i: (0, i, 0)),
                  pl.BlockSpec((1, D), lambda i: (0, 0)),
                  pl.BlockSpec((D, D), lambda i: (0, 0))],
        out_specs=pl.BlockSpec((RB, D), lambda i: (i, 0)),
        out_shape=jax.ShapeDtypeStruct((N, D), jnp.float32),
    )(p, b.reshape(1, D), M)


def kernel(x, edge_index, adj_values, W1, b1, W2, b2, W3, b3):
    dst = edge_index[0]
    src = edge_index[1]

    within = jnp.concatenate([jnp.arange(0, 32, 2), jnp.arange(1, 32, 2)])
    perm = (jnp.arange(0, D, 32)[:, None] + within[None, :]).reshape(-1)
    unperm = jax.nn.one_hot(perm, D, dtype=jnp.float32)

    def pack_i32(sup_bf16):
        return lax.bitcast_convert_type(
            sup_bf16.reshape(N, D // 2, 2), jnp.int32)

    sup1 = pack_i32(_mm_plain(x, W1))
    p1 = _sc_spmv(sup1, src, dst, adj_values)
    sup2 = pack_i32(_mm_fused(p1, b1[perm], W2[perm, :]))
    p2 = _sc_spmv(sup2, src, dst, adj_values)
    sup3 = pack_i32(_mm_fused(p2, b2[perm], W3[perm, :]))
    p3 = _sc_spmv(sup3, src, dst, adj_values)
    return _final(p3, b3, unperm)
